# baseline (speedup 1.0000x reference)
import jax
import jax.numpy as jnp
from jax.experimental import pallas as pl


def _final_linear_body(hc_ref, w_ref, b_ref, out_ref):
    out_ref[...] = hc_ref[...] @ w_ref[...] + b_ref[...]


def _gcn_conv(h, src, dst, norm, W, b):
    hw = h @ W
    msg = norm[:, None] * jnp.take(hw, src, axis=0)
    out = jnp.zeros_like(hw).at[dst].add(msg)
    return out + b


def kernel(x, edge_index, batch_index, W0, b0, W1, b1, W2, b2, W3, b3, Wout, bout):
    n = x.shape[0]
    loop = jnp.arange(n, dtype=edge_index.dtype)
    src = jnp.concatenate([edge_index[0], loop])
    dst = jnp.concatenate([edge_index[1], loop])
    deg = jnp.zeros((n,), jnp.float32).at[dst].add(1.0)
    dinv = jnp.where(deg > 0, jax.lax.rsqrt(jnp.maximum(deg, 1e-12)), 0.0)
    norm = dinv[src] * dinv[dst]

    hidden = jnp.tanh(_gcn_conv(x, src, dst, norm, W0, b0))
    hidden = jnp.tanh(_gcn_conv(hidden, src, dst, norm, W1, b1))
    hidden = jnp.tanh(_gcn_conv(hidden, src, dst, norm, W2, b2))
    hidden = jnp.tanh(_gcn_conv(hidden, src, dst, norm, W3, b3))

    B = 128
    gmp = jax.ops.segment_max(hidden, batch_index, num_segments=B)
    gmp = jnp.where(jnp.isfinite(gmp), gmp, 0.0)
    seg_sum = jax.ops.segment_sum(hidden, batch_index, num_segments=B)
    counts = jax.ops.segment_sum(jnp.ones((n,), jnp.float32), batch_index, num_segments=B)
    gap = seg_sum / jnp.maximum(counts, 1.0)[:, None]
    hidden_cat = jnp.concatenate([gmp, gap], axis=1)

    out = pl.pallas_call(
        _final_linear_body,
        out_shape=jax.ShapeDtypeStruct((B, Wout.shape[1]), jnp.float32),
    )(hidden_cat, Wout, bout[None, :])
    return (out, hidden_cat)


# speedup vs baseline: 5.2529x; 5.2529x over previous
import functools

import jax
import jax.numpy as jnp
from jax import lax
from jax.experimental import pallas as pl
from jax.experimental.pallas import tpu as pltpu
from jax.experimental.pallas import tpu_sc as plsc

N = 100000
NP = 100352
E = 1600000
EP = 1605632
EDGE_ROWS = EP // 128
ROWS_PER_SUB = 392
STAGES = 49
STRIPE = 6272
B = 128
EMB = 64
FBW = 16
NFB = 4


_sc_mesh = plsc.VectorSubcoreMesh(core_axis_name="c", subcore_axis_name="s")


@functools.partial(
    pl.kernel,
    mesh=_sc_mesh,
    compiler_params=pltpu.CompilerParams(use_tc_tiling_on_sc=False),
    out_type=jax.ShapeDtypeStruct((NFB, 2, NP, FBW), jnp.float32),
    scratch_types=[
        pltpu.VMEM((8, 128), jnp.int32),
        pltpu.VMEM((8, 128), jnp.int32),
        pltpu.VMEM((128, FBW), jnp.float32),
        pltpu.VMEM_SHARED((NP, FBW), jnp.float32),
    ],
)
def _sc_aggregate(u0, u1, u2, u3, srcp, dstp, zrows, out, sidx, didx, rows, acc):
    cid = lax.axis_index("c")
    sid = lax.axis_index("s")
    stripe0 = sid * STRIPE
    rowbase = cid * (ROWS_PER_SUB * 16) + sid * ROWS_PER_SUB

    for b, u in enumerate((u0, u1, u2, u3)):
        pltpu.sync_copy(zrows.at[pl.ds(stripe0, STRIPE)],
                        acc.at[pl.ds(stripe0, STRIPE)])
        plsc.subcore_barrier()

        def stage(g, carry):
            r0 = rowbase + g * 8
            pltpu.sync_copy(srcp.at[pl.ds(r0, 8)], sidx)
            pltpu.sync_copy(dstp.at[pl.ds(r0, 8)], didx)
            for j in range(8):
                pltpu.sync_copy(u.at[sidx.at[j]], rows)
                pltpu.sync_copy(rows, acc.at[didx.at[j]], add=True)
            return carry

        lax.fori_loop(0, STAGES, stage, 0)
        plsc.subcore_barrier()
        pltpu.sync_copy(acc.at[pl.ds(stripe0, STRIPE)],
                        out.at[b, cid, pl.ds(stripe0, STRIPE)])
        plsc.subcore_barrier()


_BR = 2048


def _mm_body(h_ref, w_ref, d_ref, o0, o1, o2, o3):
    r = jnp.dot(h_ref[...], w_ref[...],
                preferred_element_type=jnp.float32) * d_ref[...]
    o0[...] = r[:, 0:16]
    o1[...] = r[:, 16:32]
    o2[...] = r[:, 32:48]
    o3[...] = r[:, 48:64]


def _mm_scaled(h, w, dinv_col):
    k = h.shape[1]
    grid = (NP // _BR,)
    outs = [jax.ShapeDtypeStruct((NP, FBW), jnp.float32)] * NFB
    return pl.pallas_call(
        _mm_body,
        grid=grid,
        in_specs=[
            pl.BlockSpec((_BR, k), lambda i: (i, 0)),
            pl.BlockSpec((k, EMB), lambda i: (0, 0)),
            pl.BlockSpec((_BR, 1), lambda i: (i, 0)),
        ],
        out_specs=[pl.BlockSpec((_BR, FBW), lambda i: (i, 0))] * NFB,
        out_shape=outs,
    )(h, w, dinv_col)


def _post_body(agg_ref, u0, u1, u2, u3, d_ref, b_ref, out_ref):
    a = agg_ref[...]
    t = jnp.concatenate(
        [a[0, 0] + a[0, 1] + u0[...],
         a[1, 0] + a[1, 1] + u1[...],
         a[2, 0] + a[2, 1] + u2[...],
         a[3, 0] + a[3, 1] + u3[...]], axis=1)
    out_ref[...] = jnp.tanh(d_ref[...] * t + b_ref[...])


def _post(agg, ublocks, dinv_col, bias):
    grid = (NP // _BR,)
    return pl.pallas_call(
        _post_body,
        grid=grid,
        in_specs=[
            pl.BlockSpec((NFB, 2, _BR, FBW), lambda i: (0, 0, i, 0)),
            pl.BlockSpec((_BR, FBW), lambda i: (i, 0)),
            pl.BlockSpec((_BR, FBW), lambda i: (i, 0)),
            pl.BlockSpec((_BR, FBW), lambda i: (i, 0)),
            pl.BlockSpec((_BR, FBW), lambda i: (i, 0)),
            pl.BlockSpec((_BR, 1), lambda i: (i, 0)),
            pl.BlockSpec((1, EMB), lambda i: (0, 0)),
        ],
        out_specs=pl.BlockSpec((_BR, EMB), lambda i: (i, 0)),
        out_shape=jax.ShapeDtypeStruct((NP, EMB), jnp.float32),
    )(agg, *ublocks, dinv_col, bias)


def _final_linear_body(hc_ref, w_ref, b_ref, out_ref):
    out_ref[...] = hc_ref[...] @ w_ref[...] + b_ref[...]


def kernel(x, edge_index, batch_index, W0, b0, W1, b1, W2, b2, W3, b3, Wout, bout):
    src = edge_index[0]
    dst = edge_index[1]

    deg = jnp.ones((N,), jnp.float32).at[dst].add(1.0)
    dinv = lax.rsqrt(deg)
    dinv_col = jnp.pad(dinv, (0, NP - N), constant_values=1.0)[:, None]

    pad = EP - E
    srcp = jnp.concatenate([src, jnp.zeros((pad,), jnp.int32)]).reshape(EDGE_ROWS, 128)
    dstp = jnp.concatenate([dst, jnp.full((pad,), NP - 1, jnp.int32)]).reshape(EDGE_ROWS, 128)
    zrows = jnp.zeros((NP, FBW), jnp.float32)

    h = jnp.pad(x, ((0, NP - N), (0, 0)))
    for W, bb in ((W0, b0), (W1, b1), (W2, b2), (W3, b3)):
        ublocks = _mm_scaled(h, W, dinv_col)
        agg = _sc_aggregate(*ublocks, srcp, dstp, zrows)
        h = _post(agg, ublocks, dinv_col, bb[None, :])

    hidden = h[:N]
    gmp = jax.ops.segment_max(hidden, batch_index, num_segments=B,
                              indices_are_sorted=True)
    gmp = jnp.where(jnp.isfinite(gmp), gmp, 0.0)
    seg_sum = jax.ops.segment_sum(hidden, batch_index, num_segments=B,
                                  indices_are_sorted=True)
    counts = jax.ops.segment_sum(jnp.ones((N,), jnp.float32), batch_index,
                                 num_segments=B, indices_are_sorted=True)
    gap = seg_sum / jnp.maximum(counts, 1.0)[:, None]
    hidden_cat = jnp.concatenate([gmp, gap], axis=1)

    out = pl.pallas_call(
        _final_linear_body,
        out_shape=jax.ShapeDtypeStruct((B, Wout.shape[1]), jnp.float32),
    )(hidden_cat, Wout, bout[None, :])
    return (out, hidden_cat)
